# baseline (speedup 1.0000x reference)
import functools

import jax
import jax.numpy as jnp
from jax.experimental import pallas as pl
from jax.experimental.pallas import tpu as pltpu
from jax.experimental.pallas import tpu_sc as plsc

K = 8
P = 16


def _gelu_exact(x):
    return 0.5 * x * (1.0 + jax.lax.erf(x * 0.7071067811865476))


def _fold_body(Wq_r, Wk_r, Wv_r, Wg_r, Wp2_r, bq_r, bk_r, bv_r, bg_r, bp2_r,
               Wcat_r, bcat_r, Wp2c_r, bp2c_r):
    Wg = Wg_r[...]
    f32 = jnp.float32
    Wqg = jnp.dot(Wq_r[...], Wg, preferred_element_type=f32)
    Wkg = jnp.dot(Wk_r[...], Wg, preferred_element_type=f32)
    Wp2g = jnp.dot(Wp2_r[...], Wg, preferred_element_type=f32)
    Wcat_r[...] = jnp.concatenate([Wqg, Wkg, Wv_r[...]], axis=1)
    bqg = jnp.sum(bq_r[...].T * Wg, axis=0, keepdims=True) + bg_r[...]
    bkg = jnp.sum(bk_r[...].T * Wg, axis=0, keepdims=True)
    bp2g = jnp.sum(bp2_r[...].T * Wg, axis=0, keepdims=True)
    bcat_r[...] = jnp.concatenate([bqg, bkg, bv_r[...]], axis=1)
    Wp2c_r[...] = jnp.concatenate([Wp2_r[...], Wp2g], axis=1)
    bp2c_r[...] = jnp.concatenate([bp2_r[...], bp2g], axis=1)


def _fold(Wq, Wk, Wv, Wg, Wp2, bq, bk, bv, bg, bp2):
    D = Wq.shape[0]
    f32 = jnp.float32
    return pl.pallas_call(
        _fold_body,
        out_shape=(
            jax.ShapeDtypeStruct((D, 3 * D), f32),
            jax.ShapeDtypeStruct((1, 3 * D), f32),
            jax.ShapeDtypeStruct((D, 2 * D), f32),
            jax.ShapeDtypeStruct((1, 2 * D), f32),
        ),
    )(Wq, Wk, Wv, Wg, Wp2, bq.reshape(1, D), bk.reshape(1, D),
      bv.reshape(1, D), bg.reshape(1, D), bp2.reshape(1, D))


def _knn_body(posm_r, posT_r, idx_r, *, M, RK, base):
    a = posm_r[0]
    tn = posT_r[0]
    sqm = a[:, 0:1] * a[:, 0:1] + a[:, 1:2] * a[:, 1:2] + a[:, 2:3] * a[:, 2:3]
    sqn = (tn[0:1, :] * tn[0:1, :] + tn[1:2, :] * tn[1:2, :]
           + tn[2:3, :] * tn[2:3, :])
    ab = jnp.dot(a.astype(jnp.bfloat16), tn.astype(jnp.bfloat16),
                 preferred_element_type=jnp.float32)
    d2 = (sqm + sqn) - 2.0 * ab
    iota = jax.lax.broadcasted_iota(jnp.int32, (RK, M), 1).astype(jnp.float32)
    big = jnp.float32(2.0 * M)
    cols = []
    for _ in range(K):
        mn = jnp.min(d2, axis=1, keepdims=True)
        cand = jnp.where(d2 <= mn, iota, big)
        j = jnp.min(cand, axis=1, keepdims=True)
        cols.append(j)
        d2 = jnp.where(iota == j, jnp.float32(jnp.inf), d2)
    idx_r[0] = jnp.concatenate(cols, axis=1).astype(jnp.int32) + base


def _knn(posp, pospT, base, RK=256):
    _, Mc, _ = posp.shape
    M = pospT.shape[2]
    grid = (Mc // RK,)
    return pl.pallas_call(
        functools.partial(_knn_body, M=M, RK=RK, base=base),
        grid=grid,
        in_specs=[
            pl.BlockSpec((1, RK, P), lambda i: (0, i, 0)),
            pl.BlockSpec((1, P, M), lambda i: (0, 0, 0)),
        ],
        out_specs=pl.BlockSpec((1, RK, K), lambda i: (0, i, 0)),
        out_shape=jax.ShapeDtypeStruct((1, Mc, K), jnp.int32),
    )(posp, pospT)


def _proj_body(qT_r, pp_r, Wcat_r, bcat_r, qg_r, tbl_r, *, D):
    acc = jnp.dot(qT_r[0].astype(jnp.bfloat16),
                  Wcat_r[...].astype(jnp.bfloat16),
                  preferred_element_type=jnp.float32) + bcat_r[...]
    qg_r[0] = acc[:, :D]
    tbl_r[0, :, :2 * D] = acc[:, D:]
    tbl_r[0, :, 2 * D:] = pp_r[0]


def _proj(qT, posp128, Wcat, bcat, RA=512):
    B, M, D = qT.shape
    PW = posp128.shape[2]
    W2 = 2 * D + PW
    grid = (B, M // RA)
    return pl.pallas_call(
        functools.partial(_proj_body, D=D),
        grid=grid,
        in_specs=[
            pl.BlockSpec((1, RA, D), lambda b, i: (b, i, 0)),
            pl.BlockSpec((1, RA, PW), lambda b, i: (b, i, 0)),
            pl.BlockSpec((D, 3 * D), lambda b, i: (0, 0)),
            pl.BlockSpec((1, 3 * D), lambda b, i: (0, 0)),
        ],
        out_specs=[
            pl.BlockSpec((1, RA, D), lambda b, i: (b, i, 0)),
            pl.BlockSpec((1, RA, W2), lambda b, i: (b, i, 0)),
        ],
        out_shape=[
            jax.ShapeDtypeStruct((B, M, D), jnp.float32),
            jax.ShapeDtypeStruct((B, M, W2), jnp.float32),
        ],
    )(qT, posp128, Wcat, bcat)


def _gather(tbl, idx, CH=64):
    NI = idx.shape[0]
    W2 = tbl.shape[1]
    mesh = plsc.VectorSubcoreMesh(core_axis_name="c", subcore_axis_name="s")
    NC, NS = mesh.num_cores, mesh.num_subcores
    NW = NC * NS
    per_w = NI // NW
    n_ch = per_w // CH
    f32 = jnp.float32

    dt = tbl.dtype

    @functools.partial(
        pl.kernel, mesh=mesh,
        out_type=jax.ShapeDtypeStruct((NI, W2), dt),
        scratch_types=[pltpu.VMEM((CH,), jnp.int32),
                       pltpu.VMEM((CH,), jnp.int32),
                       pltpu.VMEM((CH, W2), dt),
                       pltpu.VMEM((CH, W2), dt),
                       pltpu.SemaphoreType.DMA,
                       pltpu.SemaphoreType.DMA],
    )
    def k(tbl_hbm, idx_hbm, out_hbm, idx0, idx1, buf0, buf1, sem0, sem1):
        wid = jax.lax.axis_index("s") * NC + jax.lax.axis_index("c")
        base = wid * per_w

        def issue(c, idx_v, buf, sem):
            pltpu.sync_copy(idx_hbm.at[pl.ds(base + c * CH, CH)], idx_v)
            return pltpu.async_copy(tbl_hbm.at[idx_v], buf, sem)

        issue(0, idx0, buf0, sem0)

        @pl.loop(0, n_ch, step=2)
        def _(c):
            issue(c + 1, idx1, buf1, sem1)
            pltpu.make_async_copy(tbl_hbm.at[idx0], buf0, sem0).wait()
            pltpu.sync_copy(buf0, out_hbm.at[pl.ds(base + c * CH, CH)])

            @pl.when(c + 2 < n_ch)
            def _():
                issue(c + 2, idx0, buf0, sem0)

            pltpu.make_async_copy(tbl_hbm.at[idx1], buf1, sem1).wait()
            pltpu.sync_copy(buf1, out_hbm.at[pl.ds(base + (c + 1) * CH, CH)])

    return k(tbl, idx)


def _attend_body(g_r, qg_r, posm_r, qT_r, Wp1_r, bp1_r, Wp2c_r,
                 bp2c_r, Wo_r, bo_r, out_r, *, D, R, scale):
    f32 = jnp.float32
    g = g_r[...]
    kgk = g[:, :D]
    vk = g[:, D:2 * D]
    pk = g[:, 2 * D:2 * D + P]
    pm = posm_r[0]
    pm_rep = jnp.broadcast_to(pm[:, None, :], (R, K, P)).reshape(R * K, P)
    rel = pm_rep - pk
    h = jnp.dot(rel, Wp1_r[...], preferred_element_type=f32) + bp1_r[...]
    gl = _gelu_exact(h)
    pe2 = jnp.dot(gl.astype(jnp.bfloat16), Wp2c_r[...].astype(jnp.bfloat16),
                  preferred_element_type=f32) + bp2c_r[...]
    pe = pe2[:, :D]
    peg = pe2[:, D:]
    qgb = qg_r[0]
    qg_rep = jnp.broadcast_to(qgb[:, None, :], (R, K, D)).reshape(R * K, D)
    logits = (qg_rep - kgk + peg) * scale
    mx = jnp.max(logits, axis=1, keepdims=True)
    e = jnp.exp(logits - mx)
    attn = e / jnp.sum(e, axis=1, keepdims=True)
    w = attn * (vk + pe)
    res = jnp.sum(w.reshape(R, K, D), axis=1)
    out_r[0] = (jnp.dot(res.astype(jnp.bfloat16),
                        Wo_r[...].astype(jnp.bfloat16),
                        preferred_element_type=f32)
                + bo_r[...] + qT_r[0])


def _attend(gkgv, qg, posp, qT, Wp1p, bp1, Wp2c, bp2c, Wo, bo, R=256):
    B, M, D = qg.shape
    W2 = gkgv.shape[1]
    nblk = M // R
    grid = (B, nblk)
    return pl.pallas_call(
        functools.partial(_attend_body, D=D, R=R, scale=float(D) ** -0.5),
        grid=grid,
        in_specs=[
            pl.BlockSpec((R * K, W2), lambda b, i: (b * nblk + i, 0)),
            pl.BlockSpec((1, R, D), lambda b, i: (b, i, 0)),
            pl.BlockSpec((1, R, P), lambda b, i: (b, i, 0)),
            pl.BlockSpec((1, R, D), lambda b, i: (b, i, 0)),
            pl.BlockSpec((P, D), lambda b, i: (0, 0)),
            pl.BlockSpec((1, D), lambda b, i: (0, 0)),
            pl.BlockSpec((D, 2 * D), lambda b, i: (0, 0)),
            pl.BlockSpec((1, 2 * D), lambda b, i: (0, 0)),
            pl.BlockSpec((D, D), lambda b, i: (0, 0)),
            pl.BlockSpec((1, D), lambda b, i: (0, 0)),
        ],
        out_specs=pl.BlockSpec((1, R, D), lambda b, i: (b, i, 0)),
        out_shape=jax.ShapeDtypeStruct((B, M, D), jnp.float32),
    )(gkgv, qg, posp, qT, Wp1p, bp1, Wp2c, bp2c, Wo, bo)


def kernel(query, pos, Wq, bq, Wk, bk, Wv, bv, Wp1, bp1, Wp2, bp2, Wg, bg,
           Wo, bo):
    M, B, D = query.shape
    qT = jnp.swapaxes(query, 0, 1)
    posp = jnp.pad(pos, ((0, 0), (0, 0), (0, P - 3)))
    posp128 = jnp.pad(pos, ((0, 0), (0, 0), (0, 128 - 3)))
    pospT = jnp.swapaxes(posp, 1, 2)
    Wp1p = jnp.pad(Wp1, ((0, P - 3), (0, 0)))
    bp1r = bp1.reshape(1, D)
    bor = bo.reshape(1, D)

    Wcat, bcat, Wp2c, bp2c = _fold(Wq, Wk, Wv, Wg, Wp2, bq, bk, bv, bg, bp2)
    qg, tbl = _proj(qT, posp128, Wcat, bcat)
    tbl2 = tbl.reshape(B * M, tbl.shape[2])

    NCH = 2
    Mc = M // NCH
    outs = []
    for b in range(B):
        posp_b = jax.lax.slice_in_dim(posp, b, b + 1, axis=0)
        pospT_b = jax.lax.slice_in_dim(pospT, b, b + 1, axis=0)
        rows = []
        for h in range(NCH):
            lo = h * Mc
            posp_c = jax.lax.slice_in_dim(posp_b, lo, lo + Mc, axis=1)
            idx_c = _knn(posp_c, pospT_b, base=b * M)
            g_c = _gather(tbl2, idx_c.reshape(Mc * K))
            qg_c = jax.lax.dynamic_slice(qg, (b, lo, 0), (1, Mc, D))
            qT_c = jax.lax.dynamic_slice(qT, (b, lo, 0), (1, Mc, D))
            rows.append(_attend(g_c, qg_c, posp_c, qT_c, Wp1p, bp1r,
                                Wp2c, bp2c, Wo, bor))
        outs.append(jnp.concatenate(rows, axis=1))
    out_bmd = jnp.concatenate(outs, axis=0)
    return jnp.swapaxes(out_bmd, 0, 1)


# speedup vs baseline: 1.0081x; 1.0081x over previous
import functools

import jax
import jax.numpy as jnp
from jax.experimental import pallas as pl
from jax.experimental.pallas import tpu as pltpu
from jax.experimental.pallas import tpu_sc as plsc

K = 8
P = 16


def _gelu_exact(x):
    return 0.5 * x * (1.0 + jax.lax.erf(x * 0.7071067811865476))


def _fold_body(Wq_r, Wk_r, Wv_r, Wg_r, Wp2_r, bq_r, bk_r, bv_r, bg_r, bp2_r,
               Wcat_r, bcat_r, Wp2c_r, bp2c_r):
    Wg = Wg_r[...]
    f32 = jnp.float32
    Wqg = jnp.dot(Wq_r[...], Wg, preferred_element_type=f32)
    Wkg = jnp.dot(Wk_r[...], Wg, preferred_element_type=f32)
    Wp2g = jnp.dot(Wp2_r[...], Wg, preferred_element_type=f32)
    Wcat_r[...] = jnp.concatenate([Wqg, Wkg, Wv_r[...]], axis=1)
    bqg = jnp.sum(bq_r[...].T * Wg, axis=0, keepdims=True) + bg_r[...]
    bkg = jnp.sum(bk_r[...].T * Wg, axis=0, keepdims=True)
    bp2g = jnp.sum(bp2_r[...].T * Wg, axis=0, keepdims=True)
    bcat_r[...] = jnp.concatenate([bqg, bkg, bv_r[...]], axis=1)
    Wp2c_r[...] = jnp.concatenate([Wp2_r[...], Wp2g], axis=1)
    bp2c_r[...] = jnp.concatenate([bp2_r[...], bp2g], axis=1)


def _fold(Wq, Wk, Wv, Wg, Wp2, bq, bk, bv, bg, bp2):
    D = Wq.shape[0]
    f32 = jnp.float32
    return pl.pallas_call(
        _fold_body,
        out_shape=(
            jax.ShapeDtypeStruct((D, 3 * D), f32),
            jax.ShapeDtypeStruct((1, 3 * D), f32),
            jax.ShapeDtypeStruct((D, 2 * D), f32),
            jax.ShapeDtypeStruct((1, 2 * D), f32),
        ),
    )(Wq, Wk, Wv, Wg, Wp2, bq.reshape(1, D), bk.reshape(1, D),
      bv.reshape(1, D), bg.reshape(1, D), bp2.reshape(1, D))


def _rowmin(x):
    W = x.shape[1]
    while W > 128:
        h = W // 2
        x = jnp.minimum(jax.lax.slice_in_dim(x, 0, h, axis=1),
                        jax.lax.slice_in_dim(x, h, W, axis=1))
        W = h
    return jnp.min(x, axis=1, keepdims=True)


def _knn_body(posm_r, posT_r, idx_r, *, M, RK, base):
    a = posm_r[0]
    tn = posT_r[0]
    sqm = a[:, 0:1] * a[:, 0:1] + a[:, 1:2] * a[:, 1:2] + a[:, 2:3] * a[:, 2:3]
    sqn = (tn[0:1, :] * tn[0:1, :] + tn[1:2, :] * tn[1:2, :]
           + tn[2:3, :] * tn[2:3, :])
    ab = jnp.dot(a.astype(jnp.bfloat16), tn.astype(jnp.bfloat16),
                 preferred_element_type=jnp.float32)
    d2 = (sqm + sqn) - 2.0 * ab
    iota = jax.lax.broadcasted_iota(jnp.int32, (RK, M), 1).astype(jnp.float32)
    big = jnp.float32(2.0 * M)
    cols = []
    for _ in range(K):
        mn = _rowmin(d2)
        cand = jnp.where(d2 <= mn, iota, big)
        j = _rowmin(cand)
        cols.append(j)
        d2 = jnp.where(iota == j, jnp.float32(jnp.inf), d2)
    idx_r[0] = jnp.concatenate(cols, axis=1).astype(jnp.int32) + base


def _knn(posp, pospT, base, RK=256):
    _, Mc, _ = posp.shape
    M = pospT.shape[2]
    grid = (Mc // RK,)
    return pl.pallas_call(
        functools.partial(_knn_body, M=M, RK=RK, base=base),
        grid=grid,
        in_specs=[
            pl.BlockSpec((1, RK, P), lambda i: (0, i, 0)),
            pl.BlockSpec((1, P, M), lambda i: (0, 0, 0)),
        ],
        out_specs=pl.BlockSpec((1, RK, K), lambda i: (0, i, 0)),
        out_shape=jax.ShapeDtypeStruct((1, Mc, K), jnp.int32),
    )(posp, pospT)


def _proj_body(qT_r, pp_r, Wcat_r, bcat_r, qg_r, tbl_r, *, D):
    acc = jnp.dot(qT_r[0].astype(jnp.bfloat16),
                  Wcat_r[...].astype(jnp.bfloat16),
                  preferred_element_type=jnp.float32) + bcat_r[...]
    qg_r[0] = acc[:, :D]
    tbl_r[0, :, :2 * D] = acc[:, D:]
    tbl_r[0, :, 2 * D:] = pp_r[0]


def _proj(qT, posp128, Wcat, bcat, RA=512):
    B, M, D = qT.shape
    PW = posp128.shape[2]
    W2 = 2 * D + PW
    grid = (B, M // RA)
    return pl.pallas_call(
        functools.partial(_proj_body, D=D),
        grid=grid,
        in_specs=[
            pl.BlockSpec((1, RA, D), lambda b, i: (b, i, 0)),
            pl.BlockSpec((1, RA, PW), lambda b, i: (b, i, 0)),
            pl.BlockSpec((D, 3 * D), lambda b, i: (0, 0)),
            pl.BlockSpec((1, 3 * D), lambda b, i: (0, 0)),
        ],
        out_specs=[
            pl.BlockSpec((1, RA, D), lambda b, i: (b, i, 0)),
            pl.BlockSpec((1, RA, W2), lambda b, i: (b, i, 0)),
        ],
        out_shape=[
            jax.ShapeDtypeStruct((B, M, D), jnp.float32),
            jax.ShapeDtypeStruct((B, M, W2), jnp.float32),
        ],
    )(qT, posp128, Wcat, bcat)


def _gather(tbl, idx, CH=64):
    NI = idx.shape[0]
    W2 = tbl.shape[1]
    mesh = plsc.VectorSubcoreMesh(core_axis_name="c", subcore_axis_name="s")
    NC, NS = mesh.num_cores, mesh.num_subcores
    NW = NC * NS
    per_w = NI // NW
    n_ch = per_w // CH
    f32 = jnp.float32

    dt = tbl.dtype

    @functools.partial(
        pl.kernel, mesh=mesh,
        out_type=jax.ShapeDtypeStruct((NI, W2), dt),
        scratch_types=[pltpu.VMEM((CH,), jnp.int32),
                       pltpu.VMEM((CH,), jnp.int32),
                       pltpu.VMEM((CH, W2), dt),
                       pltpu.VMEM((CH, W2), dt),
                       pltpu.SemaphoreType.DMA,
                       pltpu.SemaphoreType.DMA],
    )
    def k(tbl_hbm, idx_hbm, out_hbm, idx0, idx1, buf0, buf1, sem0, sem1):
        wid = jax.lax.axis_index("s") * NC + jax.lax.axis_index("c")
        base = wid * per_w

        def issue(c, idx_v, buf, sem):
            pltpu.sync_copy(idx_hbm.at[pl.ds(base + c * CH, CH)], idx_v)
            return pltpu.async_copy(tbl_hbm.at[idx_v], buf, sem)

        issue(0, idx0, buf0, sem0)

        @pl.loop(0, n_ch, step=2)
        def _(c):
            issue(c + 1, idx1, buf1, sem1)
            pltpu.make_async_copy(tbl_hbm.at[idx0], buf0, sem0).wait()
            pltpu.sync_copy(buf0, out_hbm.at[pl.ds(base + c * CH, CH)])

            @pl.when(c + 2 < n_ch)
            def _():
                issue(c + 2, idx0, buf0, sem0)

            pltpu.make_async_copy(tbl_hbm.at[idx1], buf1, sem1).wait()
            pltpu.sync_copy(buf1, out_hbm.at[pl.ds(base + (c + 1) * CH, CH)])

    return k(tbl, idx)


def _attend_body(g_r, qg_r, posm_r, qT_r, Wp1_r, bp1_r, Wp2c_r,
                 bp2c_r, Wo_r, bo_r, out_r, *, D, R, scale):
    f32 = jnp.float32
    g = g_r[...]
    kgk = g[:, :D]
    vk = g[:, D:2 * D]
    pk = g[:, 2 * D:2 * D + P]
    pm = posm_r[0]
    pm_rep = jnp.broadcast_to(pm[:, None, :], (R, K, P)).reshape(R * K, P)
    rel = pm_rep - pk
    h = jnp.dot(rel, Wp1_r[...], preferred_element_type=f32) + bp1_r[...]
    gl = _gelu_exact(h)
    pe2 = jnp.dot(gl.astype(jnp.bfloat16), Wp2c_r[...].astype(jnp.bfloat16),
                  preferred_element_type=f32) + bp2c_r[...]
    pe = pe2[:, :D]
    peg = pe2[:, D:]
    qgb = qg_r[0]
    qg_rep = jnp.broadcast_to(qgb[:, None, :], (R, K, D)).reshape(R * K, D)
    logits = (qg_rep - kgk + peg) * scale
    mx = jnp.max(logits, axis=1, keepdims=True)
    e = jnp.exp(logits - mx)
    attn = e / jnp.sum(e, axis=1, keepdims=True)
    w = attn * (vk + pe)
    res = jnp.sum(w.reshape(R, K, D), axis=1)
    out_r[0] = (jnp.dot(res.astype(jnp.bfloat16),
                        Wo_r[...].astype(jnp.bfloat16),
                        preferred_element_type=f32)
                + bo_r[...] + qT_r[0])


def _attend(gkgv, qg, posp, qT, Wp1p, bp1, Wp2c, bp2c, Wo, bo, R=256):
    B, M, D = qg.shape
    W2 = gkgv.shape[1]
    nblk = M // R
    grid = (B, nblk)
    return pl.pallas_call(
        functools.partial(_attend_body, D=D, R=R, scale=float(D) ** -0.5),
        grid=grid,
        in_specs=[
            pl.BlockSpec((R * K, W2), lambda b, i: (b * nblk + i, 0)),
            pl.BlockSpec((1, R, D), lambda b, i: (b, i, 0)),
            pl.BlockSpec((1, R, P), lambda b, i: (b, i, 0)),
            pl.BlockSpec((1, R, D), lambda b, i: (b, i, 0)),
            pl.BlockSpec((P, D), lambda b, i: (0, 0)),
            pl.BlockSpec((1, D), lambda b, i: (0, 0)),
            pl.BlockSpec((D, 2 * D), lambda b, i: (0, 0)),
            pl.BlockSpec((1, 2 * D), lambda b, i: (0, 0)),
            pl.BlockSpec((D, D), lambda b, i: (0, 0)),
            pl.BlockSpec((1, D), lambda b, i: (0, 0)),
        ],
        out_specs=pl.BlockSpec((1, R, D), lambda b, i: (b, i, 0)),
        out_shape=jax.ShapeDtypeStruct((B, M, D), jnp.float32),
    )(gkgv, qg, posp, qT, Wp1p, bp1, Wp2c, bp2c, Wo, bo)


def kernel(query, pos, Wq, bq, Wk, bk, Wv, bv, Wp1, bp1, Wp2, bp2, Wg, bg,
           Wo, bo):
    M, B, D = query.shape
    qT = jnp.swapaxes(query, 0, 1)
    posp = jnp.pad(pos, ((0, 0), (0, 0), (0, P - 3)))
    posp128 = jnp.pad(pos, ((0, 0), (0, 0), (0, 128 - 3)))
    pospT = jnp.swapaxes(posp, 1, 2)
    Wp1p = jnp.pad(Wp1, ((0, P - 3), (0, 0)))
    bp1r = bp1.reshape(1, D)
    bor = bo.reshape(1, D)

    Wcat, bcat, Wp2c, bp2c = _fold(Wq, Wk, Wv, Wg, Wp2, bq, bk, bv, bg, bp2)
    qg, tbl = _proj(qT, posp128, Wcat, bcat)
    tbl2 = tbl.reshape(B * M, tbl.shape[2])

    NCH = 2
    Mc = M // NCH
    outs = []
    for b in range(B):
        posp_b = jax.lax.slice_in_dim(posp, b, b + 1, axis=0)
        pospT_b = jax.lax.slice_in_dim(pospT, b, b + 1, axis=0)
        rows = []
        for h in range(NCH):
            lo = h * Mc
            posp_c = jax.lax.slice_in_dim(posp_b, lo, lo + Mc, axis=1)
            idx_c = _knn(posp_c, pospT_b, base=b * M)
            g_c = _gather(tbl2, idx_c.reshape(Mc * K))
            qg_c = jax.lax.dynamic_slice(qg, (b, lo, 0), (1, Mc, D))
            qT_c = jax.lax.dynamic_slice(qT, (b, lo, 0), (1, Mc, D))
            rows.append(_attend(g_c, qg_c, posp_c, qT_c, Wp1p, bp1r,
                                Wp2c, bp2c, Wo, bor))
        outs.append(jnp.concatenate(rows, axis=1))
    out_bmd = jnp.concatenate(outs, axis=0)
    return jnp.swapaxes(out_bmd, 0, 1)
